# baseline (speedup 1.0000x reference)
import functools

import numpy as np
import jax
import jax.numpy as jnp
from jax import lax
from jax.experimental import pallas as pl
from jax.experimental.pallas import tpu as pltpu
from jax.experimental.pallas import tpu_sc as plsc

BATCH = 32
WINDOW = 512
HOP = 128
NFFT = 512
FRAMES = 256
BINS = 257
SIG_LEN = (FRAMES - 1) * HOP + WINDOW
ROWS_PAD = 264
SIG_PAD = ROWS_PAD * HOP
NUM_SAMPLES = 2048
NUM_WORKERS = 32
CHUNK = SIG_PAD // NUM_WORKERS


def _np_hann(length):
    n = np.arange(length, dtype=np.float64)
    return 0.5 - 0.5 * np.cos(2.0 * np.pi * n / length)


def _np_inverse_stft_window(frame_length, frame_step):
    window = _np_hann(frame_length)
    denom = window ** 2
    overlaps = -(-frame_length // frame_step)
    denom = np.pad(denom, (0, overlaps * frame_step - frame_length))
    denom = denom.reshape(overlaps, frame_step).sum(axis=0)
    denom = np.tile(denom, overlaps)[:frame_length]
    return window / denom


def _build_mats():
    k = np.arange(BINS, dtype=np.float64)
    l = np.arange(WINDOW, dtype=np.float64)
    ang = 2.0 * np.pi * np.outer(k, l) / NFFT
    a = np.full(BINS, 2.0)
    a[0] = 1.0
    a[BINS - 1] = 1.0
    winv = _np_inverse_stft_window(WINDOW, HOP)
    m1r = ((a[:, None] * np.cos(ang)) / NFFT) * winv[None, :]
    m1i = ((-a[:, None] * np.sin(ang)) / NFFT) * winv[None, :]
    h = _np_hann(WINDOW)
    m2r = np.cos(ang).T * h[:, None]
    m2i = -np.sin(ang).T * h[:, None]
    m2cat = np.concatenate([m2r, m2i], axis=1)
    return (np.asarray(m1r, np.float32), np.asarray(m1i, np.float32),
            np.asarray(m2cat, np.float32))


_M1R_NP, _M1I_NP, _M2CAT_NP = _build_mats()
_NB = 4


def _build_mask(cut_indices):
    mesh = plsc.VectorSubcoreMesh(core_axis_name="c", subcore_axis_name="s",
                                  num_cores=1)
    n_sub = 16
    chunk = SIG_PAD // n_sub
    idx_per = NUM_SAMPLES // n_sub

    @functools.partial(
        pl.kernel,
        mesh=mesh,
        out_type=jax.ShapeDtypeStruct((SIG_PAD,), jnp.float32),
        scratch_types=[
            pltpu.VMEM((chunk,), jnp.float32),
            pltpu.VMEM((n_sub, idx_per), jnp.int32),
            pltpu.VMEM((idx_per,), jnp.float32),
            pltpu.SemaphoreType.DMA,
        ],
    )
    def mask_kernel(cut_hbm, ones_hbm, zeros_hbm, out_hbm,
                    chunk_v, idx_v, zeros_v, sem):
        sid = lax.axis_index("s")
        base = sid * chunk
        pltpu.sync_copy(ones_hbm, chunk_v)
        pltpu.sync_copy(chunk_v, out_hbm.at[pl.ds(base, chunk)])
        pltpu.sync_copy(cut_hbm, idx_v)
        pltpu.sync_copy(zeros_hbm, zeros_v)
        plsc.subcore_barrier()
        pltpu.async_copy(zeros_v, out_hbm.at[idx_v.at[sid]], sem).wait()

    ones = jnp.ones((chunk,), jnp.float32)
    zeros = jnp.zeros((idx_per,), jnp.float32)
    return mask_kernel(cut_indices.reshape(n_sub, idx_per), ones, zeros)


_C00 = (((0,), (0,)), ((), ()))
_C01 = (((0,), (1,)), ((), ()))


def _tc_body(x_ref, m1r_ref, m1i_ref, m2_ref, mask_ref, o_ref):
    re = jnp.concatenate([x_ref[i, :, 0, :] for i in range(_NB)], axis=1)
    im = jnp.concatenate([x_ref[i, :, 1, :] for i in range(_NB)], axis=1)
    fi = (lax.dot_general(re.astype(jnp.bfloat16), m1r_ref[...], _C00,
                          preferred_element_type=jnp.float32) +
          lax.dot_general(im.astype(jnp.bfloat16), m1i_ref[...], _C00,
                          preferred_element_type=jnp.float32))
    ys = []
    for i in range(_NB):
        fib = fi[FRAMES * i:FRAMES * (i + 1)]
        parts = []
        for j in range(4):
            seg = fib[:, 128 * j:128 * (j + 1)]
            pieces = [seg, jnp.zeros((8 - j, 128), jnp.float32)]
            if j:
                pieces.insert(0, jnp.zeros((j, 128), jnp.float32))
            parts.append(jnp.concatenate(pieces, axis=0))
        s = parts[0] + parts[1] + parts[2] + parts[3]
        z = s * mask_ref[...]
        ys.append(jnp.concatenate([z[j:j + 256, :] for j in range(4)], axis=1))
    y = jnp.concatenate(ys, axis=0).astype(jnp.bfloat16)
    ore = lax.dot_general(m2_ref[:, :BINS], y, _C01,
                          preferred_element_type=jnp.float32)
    oim = lax.dot_general(m2_ref[:, BINS:], y, _C01,
                          preferred_element_type=jnp.float32)
    for i in range(_NB):
        o_ref[i, :, 0, :] = ore[:, FRAMES * i:FRAMES * (i + 1)]
        o_ref[i, :, 1, :] = oim[:, FRAMES * i:FRAMES * (i + 1)]


def _tc_pipeline(x, m1r, m1i, m2, mask2d):
    return pl.pallas_call(
        _tc_body,
        grid=(BATCH // _NB,),
        in_specs=[
            pl.BlockSpec((_NB, BINS, 2, FRAMES), lambda b: (b, 0, 0, 0)),
            pl.BlockSpec((BINS, WINDOW), lambda b: (0, 0)),
            pl.BlockSpec((BINS, WINDOW), lambda b: (0, 0)),
            pl.BlockSpec((WINDOW, 2 * BINS), lambda b: (0, 0)),
            pl.BlockSpec((ROWS_PAD, HOP), lambda b: (0, 0)),
        ],
        out_specs=pl.BlockSpec((_NB, BINS, 2, FRAMES), lambda b: (b, 0, 0, 0)),
        out_shape=jax.ShapeDtypeStruct((BATCH, BINS, 2, FRAMES), jnp.float32),
    )(x, m1r, m1i, m2, mask2d)


def kernel(inputs, cut_indices):
    x = inputs.transpose(0, 1, 3, 2)
    mask = _build_mask(cut_indices)
    o = _tc_pipeline(x, jnp.asarray(_M1R_NP, jnp.bfloat16),
                     jnp.asarray(_M1I_NP, jnp.bfloat16),
                     jnp.asarray(_M2CAT_NP, jnp.bfloat16),
                     mask.reshape(ROWS_PAD, HOP))
    return o.transpose(0, 1, 3, 2)


# speedup vs baseline: 1.0970x; 1.0970x over previous
import functools

import numpy as np
import jax
import jax.numpy as jnp
from jax import lax
from jax.experimental import pallas as pl
from jax.experimental.pallas import tpu as pltpu
from jax.experimental.pallas import tpu_sc as plsc

BATCH = 32
WINDOW = 512
HOP = 128
NFFT = 512
FRAMES = 256
BINS = 257
SIG_LEN = (FRAMES - 1) * HOP + WINDOW
ROWS_PAD = 264
SIG_PAD = ROWS_PAD * HOP
NUM_SAMPLES = 2048
NUM_WORKERS = 32
CHUNK = SIG_PAD // NUM_WORKERS


def _np_hann(length):
    n = np.arange(length, dtype=np.float64)
    return 0.5 - 0.5 * np.cos(2.0 * np.pi * n / length)


def _np_inverse_stft_window(frame_length, frame_step):
    window = _np_hann(frame_length)
    denom = window ** 2
    overlaps = -(-frame_length // frame_step)
    denom = np.pad(denom, (0, overlaps * frame_step - frame_length))
    denom = denom.reshape(overlaps, frame_step).sum(axis=0)
    denom = np.tile(denom, overlaps)[:frame_length]
    return window / denom


def _build_mats():
    k = np.arange(BINS, dtype=np.float64)
    l = np.arange(WINDOW, dtype=np.float64)
    ang = 2.0 * np.pi * np.outer(k, l) / NFFT
    a = np.full(BINS, 2.0)
    a[0] = 1.0
    a[BINS - 1] = 1.0
    winv = _np_inverse_stft_window(WINDOW, HOP)
    m1r = ((a[:, None] * np.cos(ang)) / NFFT) * winv[None, :]
    m1i = ((-a[:, None] * np.sin(ang)) / NFFT) * winv[None, :]
    h = _np_hann(WINDOW)
    m2r = np.cos(ang).T * h[:, None]
    m2i = -np.sin(ang).T * h[:, None]
    m2cat = np.concatenate([m2r, m2i], axis=1)
    return (np.asarray(m1r, np.float32), np.asarray(m1i, np.float32),
            np.asarray(m2cat, np.float32))


_M1R_NP, _M1I_NP, _M2CAT_NP = _build_mats()
_NB = 4


def _build_mask(cut_indices):
    mesh = plsc.VectorSubcoreMesh(core_axis_name="c", subcore_axis_name="s",
                                  num_cores=1)
    n_sub = 16
    chunk = SIG_PAD // n_sub
    idx_per = NUM_SAMPLES // n_sub

    @functools.partial(
        pl.kernel,
        mesh=mesh,
        out_type=jax.ShapeDtypeStruct((SIG_PAD,), jnp.float32),
        scratch_types=[
            pltpu.VMEM((chunk,), jnp.float32),
            pltpu.VMEM((n_sub, idx_per), jnp.int32),
            pltpu.VMEM((idx_per,), jnp.float32),
            pltpu.SemaphoreType.DMA,
        ],
    )
    def mask_kernel(cut_hbm, ones_hbm, zeros_hbm, out_hbm,
                    chunk_v, idx_v, zeros_v, sem):
        sid = lax.axis_index("s")
        base = sid * chunk
        pltpu.sync_copy(ones_hbm, chunk_v)
        pltpu.sync_copy(chunk_v, out_hbm.at[pl.ds(base, chunk)])
        pltpu.sync_copy(cut_hbm, idx_v)
        pltpu.sync_copy(zeros_hbm, zeros_v)
        plsc.subcore_barrier()
        pltpu.async_copy(zeros_v, out_hbm.at[idx_v.at[sid]], sem).wait()

    ones = jnp.ones((chunk,), jnp.float32)
    zeros = jnp.zeros((idx_per,), jnp.float32)
    return mask_kernel(cut_indices.reshape(n_sub, idx_per), ones, zeros)


_C00 = (((0,), (0,)), ((), ()))
_C01 = (((0,), (1,)), ((), ()))


def _tc1_body(x_ref, m1r_ref, m1i_ref, o_ref):
    re = jnp.concatenate([x_ref[i, :, 0, :] for i in range(_NB)], axis=1)
    im = jnp.concatenate([x_ref[i, :, 1, :] for i in range(_NB)], axis=1)
    fi = (lax.dot_general(re.astype(jnp.bfloat16), m1r_ref[...], _C00,
                          preferred_element_type=jnp.float32) +
          lax.dot_general(im.astype(jnp.bfloat16), m1i_ref[...], _C00,
                          preferred_element_type=jnp.float32))
    for i in range(_NB):
        fib = fi[FRAMES * i:FRAMES * (i + 1)]
        parts = []
        for j in range(4):
            seg = fib[:, 128 * j:128 * (j + 1)]
            pieces = [seg, jnp.zeros((8 - j, 128), jnp.float32)]
            if j:
                pieces.insert(0, jnp.zeros((j, 128), jnp.float32))
            parts.append(jnp.concatenate(pieces, axis=0))
        o_ref[i] = parts[0] + parts[1] + parts[2] + parts[3]


def _tc2_body(s_ref, m2_ref, mask_ref, o_ref):
    ys = []
    for i in range(_NB):
        z = s_ref[i] * mask_ref[...]
        ys.append(jnp.concatenate([z[j:j + 256, :] for j in range(4)], axis=1))
    y = jnp.concatenate(ys, axis=0).astype(jnp.bfloat16)
    ore = lax.dot_general(m2_ref[:, :BINS], y, _C01,
                          preferred_element_type=jnp.float32)
    oim = lax.dot_general(m2_ref[:, BINS:], y, _C01,
                          preferred_element_type=jnp.float32)
    for i in range(_NB):
        o_ref[i, :, 0, :] = ore[:, FRAMES * i:FRAMES * (i + 1)]
        o_ref[i, :, 1, :] = oim[:, FRAMES * i:FRAMES * (i + 1)]


def _tc_pipeline(x, m1r, m1i, m2, mask2d):
    s_all = pl.pallas_call(
        _tc1_body,
        grid=(BATCH // _NB,),
        in_specs=[
            pl.BlockSpec((_NB, BINS, 2, FRAMES), lambda b: (b, 0, 0, 0)),
            pl.BlockSpec((BINS, WINDOW), lambda b: (0, 0)),
            pl.BlockSpec((BINS, WINDOW), lambda b: (0, 0)),
        ],
        out_specs=pl.BlockSpec((_NB, ROWS_PAD, HOP), lambda b: (b, 0, 0)),
        out_shape=jax.ShapeDtypeStruct((BATCH, ROWS_PAD, HOP), jnp.float32),
    )(x, m1r, m1i)
    return pl.pallas_call(
        _tc2_body,
        grid=(BATCH // _NB,),
        in_specs=[
            pl.BlockSpec((_NB, ROWS_PAD, HOP), lambda b: (b, 0, 0)),
            pl.BlockSpec((WINDOW, 2 * BINS), lambda b: (0, 0)),
            pl.BlockSpec((ROWS_PAD, HOP), lambda b: (0, 0)),
        ],
        out_specs=pl.BlockSpec((_NB, BINS, 2, FRAMES), lambda b: (b, 0, 0, 0)),
        out_shape=jax.ShapeDtypeStruct((BATCH, BINS, 2, FRAMES), jnp.float32),
    )(s_all, m2, mask2d)


def kernel(inputs, cut_indices):
    x = inputs.transpose(0, 1, 3, 2)
    mask = _build_mask(cut_indices)
    o = _tc_pipeline(x, jnp.asarray(_M1R_NP, jnp.bfloat16),
                     jnp.asarray(_M1I_NP, jnp.bfloat16),
                     jnp.asarray(_M2CAT_NP, jnp.bfloat16),
                     mask.reshape(ROWS_PAD, HOP))
    return o.transpose(0, 1, 3, 2)


# speedup vs baseline: 1.1170x; 1.0182x over previous
import functools

import numpy as np
import jax
import jax.numpy as jnp
from jax import lax
from jax.experimental import pallas as pl
from jax.experimental.pallas import tpu as pltpu
from jax.experimental.pallas import tpu_sc as plsc

BATCH = 32
WINDOW = 512
HOP = 128
NFFT = 512
FRAMES = 256
BINS = 257
SIG_LEN = (FRAMES - 1) * HOP + WINDOW
ROWS_PAD = 264
SIG_PAD = ROWS_PAD * HOP
NUM_SAMPLES = 2048
NUM_WORKERS = 32
CHUNK = SIG_PAD // NUM_WORKERS


def _np_hann(length):
    n = np.arange(length, dtype=np.float64)
    return 0.5 - 0.5 * np.cos(2.0 * np.pi * n / length)


def _np_inverse_stft_window(frame_length, frame_step):
    window = _np_hann(frame_length)
    denom = window ** 2
    overlaps = -(-frame_length // frame_step)
    denom = np.pad(denom, (0, overlaps * frame_step - frame_length))
    denom = denom.reshape(overlaps, frame_step).sum(axis=0)
    denom = np.tile(denom, overlaps)[:frame_length]
    return window / denom


def _build_mats():
    k = np.arange(BINS, dtype=np.float64)
    l = np.arange(WINDOW, dtype=np.float64)
    ang = 2.0 * np.pi * np.outer(k, l) / NFFT
    a = np.full(BINS, 2.0)
    a[0] = 1.0
    a[BINS - 1] = 1.0
    winv = _np_inverse_stft_window(WINDOW, HOP)
    m1r = ((a[:, None] * np.cos(ang)) / NFFT) * winv[None, :]
    m1i = ((-a[:, None] * np.sin(ang)) / NFFT) * winv[None, :]
    h = _np_hann(WINDOW)
    m2r = np.cos(ang).T * h[:, None]
    m2i = -np.sin(ang).T * h[:, None]
    m2cat = np.concatenate([m2r, m2i], axis=1)
    return (np.asarray(m1r, np.float32), np.asarray(m1i, np.float32),
            np.asarray(m2cat, np.float32))


_M1R_NP, _M1I_NP, _M2CAT_NP = _build_mats()
_NB = 8


def _build_mask(cut_indices):
    mesh = plsc.VectorSubcoreMesh(core_axis_name="c", subcore_axis_name="s",
                                  num_cores=1)
    n_sub = 16
    chunk = SIG_PAD // n_sub
    idx_per = NUM_SAMPLES // n_sub

    @functools.partial(
        pl.kernel,
        mesh=mesh,
        out_type=jax.ShapeDtypeStruct((SIG_PAD,), jnp.float32),
        scratch_types=[
            pltpu.VMEM((chunk,), jnp.float32),
            pltpu.VMEM((n_sub, idx_per), jnp.int32),
            pltpu.VMEM((idx_per,), jnp.float32),
            pltpu.SemaphoreType.DMA,
        ],
    )
    def mask_kernel(cut_hbm, ones_hbm, zeros_hbm, out_hbm,
                    chunk_v, idx_v, zeros_v, sem):
        sid = lax.axis_index("s")
        base = sid * chunk
        pltpu.sync_copy(ones_hbm, chunk_v)
        pltpu.sync_copy(chunk_v, out_hbm.at[pl.ds(base, chunk)])
        pltpu.sync_copy(cut_hbm, idx_v)
        pltpu.sync_copy(zeros_hbm, zeros_v)
        plsc.subcore_barrier()
        pltpu.async_copy(zeros_v, out_hbm.at[idx_v.at[sid]], sem).wait()

    ones = jnp.ones((chunk,), jnp.float32)
    zeros = jnp.zeros((idx_per,), jnp.float32)
    return mask_kernel(cut_indices.reshape(n_sub, idx_per), ones, zeros)


_C00 = (((0,), (0,)), ((), ()))
_C01 = (((0,), (1,)), ((), ()))


def _tc1_body(x_ref, m1r_ref, m1i_ref, o_ref):
    re = jnp.concatenate([x_ref[i, :, 0, :] for i in range(_NB)], axis=1)
    im = jnp.concatenate([x_ref[i, :, 1, :] for i in range(_NB)], axis=1)
    fi = (lax.dot_general(re.astype(jnp.bfloat16), m1r_ref[...], _C00,
                          preferred_element_type=jnp.float32) +
          lax.dot_general(im.astype(jnp.bfloat16), m1i_ref[...], _C00,
                          preferred_element_type=jnp.float32))
    for i in range(_NB):
        fib = fi[FRAMES * i:FRAMES * (i + 1)]
        parts = []
        for j in range(4):
            seg = fib[:, 128 * j:128 * (j + 1)]
            pieces = [seg, jnp.zeros((8 - j, 128), jnp.float32)]
            if j:
                pieces.insert(0, jnp.zeros((j, 128), jnp.float32))
            parts.append(jnp.concatenate(pieces, axis=0))
        o_ref[i] = parts[0] + parts[1] + parts[2] + parts[3]


def _tc2_body(s_ref, m2_ref, mask_ref, o_ref):
    ys = []
    for i in range(_NB):
        z = s_ref[i] * mask_ref[...]
        ys.append(jnp.concatenate([z[j:j + 256, :] for j in range(4)], axis=1))
    y = jnp.concatenate(ys, axis=0).astype(jnp.bfloat16)
    ore = lax.dot_general(m2_ref[:, :BINS], y, _C01,
                          preferred_element_type=jnp.float32)
    oim = lax.dot_general(m2_ref[:, BINS:], y, _C01,
                          preferred_element_type=jnp.float32)
    for i in range(_NB):
        o_ref[i, :, 0, :] = ore[:, FRAMES * i:FRAMES * (i + 1)]
        o_ref[i, :, 1, :] = oim[:, FRAMES * i:FRAMES * (i + 1)]


def _tc_pipeline(x, m1r, m1i, m2, mask2d):
    s_all = pl.pallas_call(
        _tc1_body,
        grid=(BATCH // _NB,),
        in_specs=[
            pl.BlockSpec((_NB, BINS, 2, FRAMES), lambda b: (b, 0, 0, 0)),
            pl.BlockSpec((BINS, WINDOW), lambda b: (0, 0)),
            pl.BlockSpec((BINS, WINDOW), lambda b: (0, 0)),
        ],
        out_specs=pl.BlockSpec((_NB, ROWS_PAD, HOP), lambda b: (b, 0, 0)),
        out_shape=jax.ShapeDtypeStruct((BATCH, ROWS_PAD, HOP), jnp.float32),
    )(x, m1r, m1i)
    return pl.pallas_call(
        _tc2_body,
        grid=(BATCH // _NB,),
        in_specs=[
            pl.BlockSpec((_NB, ROWS_PAD, HOP), lambda b: (b, 0, 0)),
            pl.BlockSpec((WINDOW, 2 * BINS), lambda b: (0, 0)),
            pl.BlockSpec((ROWS_PAD, HOP), lambda b: (0, 0)),
        ],
        out_specs=pl.BlockSpec((_NB, BINS, 2, FRAMES), lambda b: (b, 0, 0, 0)),
        out_shape=jax.ShapeDtypeStruct((BATCH, BINS, 2, FRAMES), jnp.float32),
    )(s_all, m2, mask2d)


def kernel(inputs, cut_indices):
    x = inputs.transpose(0, 1, 3, 2)
    mask = _build_mask(cut_indices)
    o = _tc_pipeline(x, jnp.asarray(_M1R_NP, jnp.bfloat16),
                     jnp.asarray(_M1I_NP, jnp.bfloat16),
                     jnp.asarray(_M2CAT_NP, jnp.bfloat16),
                     mask.reshape(ROWS_PAD, HOP))
    return o.transpose(0, 1, 3, 2)
